# baseline (speedup 1.0000x reference)
import jax
import jax.numpy as jnp
from jax import lax
from jax.experimental import pallas as pl
from jax.experimental.pallas import tpu as pltpu
from jax.experimental.pallas import tpu_sc as plsc

_N = 10000
_D = 128
_E = 320000
_B = 16384
_NP = 10240
_NCH = 2560
_TCH = _NCH // 32
_RPT = _NP // 16
_RB = 2560


def _mesh():
    return plsc.VectorSubcoreMesh(
        core_axis_name="c", subcore_axis_name="s", num_cores=2, num_subcores=16
    )


def _deg_body(dstc, zvec, onev, out, idx_d, zbuf, obuf, acc):
    cid = lax.axis_index("c")
    sid = lax.axis_index("s")
    wid = sid * 2 + cid
    pltpu.sync_copy(zvec, zbuf)
    pltpu.sync_copy(zbuf, acc.at[pl.ds(sid * _RPT, _RPT)])
    pltpu.sync_copy(onev, obuf)
    plsc.subcore_barrier()
    pltpu.sync_copy(dstc.at[pl.ds(wid * _TCH, _TCH)], idx_d)

    def step(c, carry):
        pltpu.sync_copy(obuf, acc.at[idx_d.at[c]], add=True)
        return carry

    lax.fori_loop(0, _TCH, step, 0)
    plsc.subcore_barrier()
    pltpu.sync_copy(acc.at[pl.ds(sid * _RPT, _RPT)],
                    out.at[cid, pl.ds(sid * _RPT, _RPT)])


def _deg(dstc, zvec, onev):
    return pl.kernel(
        _deg_body,
        out_type=jax.ShapeDtypeStruct((2, _NP), jnp.float32),
        mesh=_mesh(),
        scratch_types=[
            pltpu.VMEM((_TCH, 128), jnp.int32),
            pltpu.VMEM((_RPT,), jnp.float32),
            pltpu.VMEM((128,), jnp.float32),
            pltpu.VMEM_SHARED((_NP,), jnp.float32),
        ],
    )(dstc, zvec, onev)


_PCH = 32
_PTC = 320
_PHH = 40
_PNB = 8
_PPD = 4


def _prop_body(table, srcc, dstc, zrows, out, idx_s, idx_d, *rest):
    bufs = rest[:_PNB]
    acc = rest[_PNB]
    sems = rest[_PNB + 1:2 * _PNB + 1]
    sss = rest[2 * _PNB + 1:]
    cid = lax.axis_index("c")
    sid = lax.axis_index("s")
    wid = sid * 2 + cid
    pltpu.sync_copy(zrows.at[pl.ds(0, _PCH)], bufs[0])
    for k in range(_RPT // _PCH):
        pltpu.sync_copy(bufs[0], acc.at[pl.ds(sid * _RPT + k * _PCH, _PCH)])
    plsc.subcore_barrier()
    for h in range(_PTC // _PHH):
        base = wid * _PTC + h * _PHH
        pltpu.sync_copy(srcc.at[pl.ds(base, _PHH)], idx_s)
        pltpu.sync_copy(dstc.at[pl.ds(base, _PHH)], idx_d)
        for b in range(_PPD):
            pltpu.async_copy(table.at[idx_s.at[b]], bufs[b], sems[b])

        def step(g, carry):
            cb = _PNB * g
            for b in range(_PNB):
                c = cb + b
                pltpu.make_async_copy(
                    table.at[idx_s.at[0]], bufs[b], sems[b]).wait()
                pltpu.async_copy(bufs[b], acc.at[idx_d.at[c]], sss[b],
                                 add=True)
                b2 = (b + _PPD) % _PNB

                @pl.when(c + _PPD < _PHH)
                def _(b2=b2, c=c):
                    @pl.when(c >= _PPD)
                    def _():
                        pltpu.make_async_copy(
                            bufs[b2], acc.at[idx_d.at[0]], sss[b2]).wait()

                    pltpu.async_copy(table.at[idx_s.at[c + _PPD]],
                                     bufs[b2], sems[b2])

            return carry

        lax.fori_loop(0, _PHH // _PNB, step, 0)
        for b in range(_PNB):
            pltpu.make_async_copy(bufs[b], acc.at[idx_d.at[0]], sss[b]).wait()
    plsc.subcore_barrier()
    pltpu.sync_copy(acc.at[pl.ds(sid * _RPT, _RPT)],
                    out.at[cid, pl.ds(sid * _RPT, _RPT)])


def _prop(table, srcc, dstc, zrows):
    return pl.kernel(
        _prop_body,
        out_type=jax.ShapeDtypeStruct((2, _NP, _D), jnp.float32),
        mesh=_mesh(),
        scratch_types=(
            [pltpu.VMEM((_PHH, _PCH), jnp.int32),
             pltpu.VMEM((_PHH, _PCH), jnp.int32)]
            + [pltpu.VMEM((_PCH, _D), jnp.float32)] * _PNB
            + [pltpu.VMEM_SHARED((_NP, _D), jnp.float32)]
            + [pltpu.SemaphoreType.DMA] * (2 * _PNB)
        ),
    )(table, srcc, dstc, zrows)


def _qg_body(zemb, qa, qb, outa, outb, idxbuf, rows, sem):
    cid = lax.axis_index("c")
    sid = lax.axis_index("s")
    wid = sid * 2 + cid
    for qref, oref in ((qa, outa), (qb, outb)):
        pltpu.sync_copy(qref.at[pl.ds(wid * 4, 4)], idxbuf)

        def step(j, carry, oref=oref):
            pltpu.async_copy(zemb.at[idxbuf.at[j]], rows, sem).wait()
            pltpu.sync_copy(rows, oref.at[pl.ds(wid * 512 + j * 128, 128)])
            return carry

        lax.fori_loop(0, 4, step, 0)


def _qgather(zemb, qa, qb):
    return pl.kernel(
        _qg_body,
        out_type=(jax.ShapeDtypeStruct((_B, _D), jnp.float32),
                  jax.ShapeDtypeStruct((_B, _D), jnp.float32)),
        mesh=_mesh(),
        scratch_types=[
            pltpu.VMEM((4, 128), jnp.int32),
            pltpu.VMEM((128, _D), jnp.float32),
            pltpu.SemaphoreType.DMA,
        ],
    )(zemb, qa, qb)


def _tcA_body(emb_ref, p0_ref, p1_ref, w1_ref, b1_ref, w2_ref, b2_ref,
              u0_ref, dinv_ref, z2_ref):
    emb = emb_ref[...]
    deg = p0_ref[...] + p1_ref[...]
    dinv = jnp.where(deg > 0, lax.rsqrt(jnp.maximum(deg, 1e-12)), 0.0)
    u0_ref[...] = emb * dinv
    dinv_ref[...] = dinv
    h = jnp.maximum(
        jnp.dot(emb, w1_ref[...], preferred_element_type=jnp.float32)
        + b1_ref[...], 0.0)
    z2_ref[...] = jnp.maximum(
        jnp.dot(h, w2_ref[...], preferred_element_type=jnp.float32)
        + b2_ref[...], 0.0)


def _tcA(emb_p, p0, p1, w1, b1, w2, b2):
    grid = _NP // _RB
    row = pl.BlockSpec((_RB, _D), lambda i: (i, 0))
    col1 = pl.BlockSpec((_RB, 1), lambda i: (i, 0))
    wsp = pl.BlockSpec((_D, _D), lambda i: (0, 0))
    bsp = pl.BlockSpec((_D,), lambda i: (0,))
    return pl.pallas_call(
        _tcA_body,
        grid=(grid,),
        in_specs=[row, col1, col1, wsp, bsp, wsp, bsp],
        out_specs=[row, col1, row],
        out_shape=[jax.ShapeDtypeStruct((_NP, _D), jnp.float32),
                   jax.ShapeDtypeStruct((_NP, 1), jnp.float32),
                   jax.ShapeDtypeStruct((_NP, _D), jnp.float32)],
    )(emb_p, p0, p1, w1, b1, w2, b2)


def _tcB_body(t1a_ref, t1b_ref, dinv_ref, u1_ref, x1_ref):
    d = dinv_ref[...]
    x1 = d * (t1a_ref[...] + t1b_ref[...])
    x1_ref[...] = x1
    u1_ref[...] = d * x1


def _tcB(t1a, t1b, dinv):
    grid = _NP // _RB
    row = pl.BlockSpec((_RB, _D), lambda i: (i, 0))
    col1 = pl.BlockSpec((_RB, 1), lambda i: (i, 0))
    return pl.pallas_call(
        _tcB_body,
        grid=(grid,),
        in_specs=[row, row, col1],
        out_specs=[row, row],
        out_shape=[jax.ShapeDtypeStruct((_NP, _D), jnp.float32),
                   jax.ShapeDtypeStruct((_NP, _D), jnp.float32)],
    )(t1a, t1b, dinv)


def _tcC_body(emb_ref, x1_ref, t2a_ref, t2b_ref, dinv_ref, z2_ref,
              wa1_ref, ba1_ref, wa2_ref, ba2_ref, zemb_ref):
    d = dinv_ref[...]
    x2 = d * (t2a_ref[...] + t2b_ref[...])
    zp = (emb_ref[...] + x1_ref[...] + x2) * (1.0 / 3.0)
    z2 = z2_ref[...]
    wa1 = wa1_ref[...]
    ba1 = ba1_ref[...]
    wa2 = wa2_ref[...]
    ba2 = ba2_ref[...]
    s1 = jnp.dot(jnp.tanh(
        jnp.dot(zp, wa1, preferred_element_type=jnp.float32) + ba1),
        wa2, preferred_element_type=jnp.float32) + ba2
    s2 = jnp.dot(jnp.tanh(
        jnp.dot(z2, wa1, preferred_element_type=jnp.float32) + ba1),
        wa2, preferred_element_type=jnp.float32) + ba2
    s10 = s1[:, 0:1]
    s11 = s1[:, 1:2]
    s20 = s2[:, 0:1]
    s21 = s2[:, 1:2]
    m = jnp.maximum(jnp.maximum(s10, s11), jnp.maximum(s20, s21))
    e10 = jnp.exp(s10 - m)
    e11 = jnp.exp(s11 - m)
    e20 = jnp.exp(s20 - m)
    e21 = jnp.exp(s21 - m)
    tot = e10 + e11 + e20 + e21
    zemb_ref[...] = (e10 * zp + e11 * z2) / tot


def _tcC(emb_p, x1, t2a, t2b, dinv, z2, wa1, ba1, wa2p, ba2p):
    grid = _NP // _RB
    row = pl.BlockSpec((_RB, _D), lambda i: (i, 0))
    col1 = pl.BlockSpec((_RB, 1), lambda i: (i, 0))
    wsp = pl.BlockSpec((_D, _D), lambda i: (0, 0))
    bsp = pl.BlockSpec((_D,), lambda i: (0,))
    return pl.pallas_call(
        _tcC_body,
        grid=(grid,),
        in_specs=[row, row, row, row, col1, row, wsp, bsp, wsp, bsp],
        out_specs=row,
        out_shape=jax.ShapeDtypeStruct((_NP, _D), jnp.float32),
    )(emb_p, x1, t2a, t2b, dinv, z2, wa1, ba1, wa2p, ba2p)


def _tcD_body(a_ref, b_ref, o_ref):
    o_ref[...] = jnp.sum(a_ref[...] * b_ref[...], axis=1, keepdims=True)


def _tcD(ga, gb):
    rb = 2048
    grid = _B // rb
    row = pl.BlockSpec((rb, _D), lambda i: (i, 0))
    col1 = pl.BlockSpec((rb, 1), lambda i: (i, 0))
    return pl.pallas_call(
        _tcD_body,
        grid=(grid,),
        in_specs=[row, row],
        out_specs=col1,
        out_shape=jax.ShapeDtypeStruct((_B, 1), jnp.float32),
    )(ga, gb)


def kernel(edge_index, edge_label_index, emb, W1, b1, W2, b2, Wa1, ba1, Wa2, ba2):
    src = edge_index[0].astype(jnp.int32)
    dst = edge_index[1].astype(jnp.int32)
    pad = jnp.full((_NCH * 128 - _E,), _N, jnp.int32)
    srcc = jnp.concatenate([src, pad]).reshape(_NCH, 128)
    dstc = jnp.concatenate([dst, pad]).reshape(_NCH, 128)
    srcp = srcc.reshape(_NCH * 128 // _PCH, _PCH)
    dstp = dstc.reshape(_NCH * 128 // _PCH, _PCH)
    emb_p = jnp.pad(emb, ((0, _NP - _N), (0, 0)))
    zrows = jnp.zeros((128, _D), jnp.float32)
    zvec = jnp.zeros((_RPT,), jnp.float32)
    onev = jnp.ones((128,), jnp.float32)

    degp = _deg(dstc, zvec, onev)
    p0 = degp[0][:, None]
    p1 = degp[1][:, None]
    u0, dinv, z2 = _tcA(emb_p, p0, p1, W1, b1, W2, b2)
    t1 = _prop(u0, srcp, dstp, zrows)
    u1, x1 = _tcB(t1[0], t1[1], dinv)
    t2 = _prop(u1, srcp, dstp, zrows)
    wa2p = jnp.pad(Wa2, ((0, 0), (0, 126)))
    ba2p = jnp.pad(ba2, ((0, 126),))
    zemb = _tcC(emb_p, x1, t2[0], t2[1], dinv, z2, Wa1, ba1, wa2p, ba2p)
    qa = edge_label_index[0].astype(jnp.int32).reshape(128, 128)
    qb = edge_label_index[1].astype(jnp.int32).reshape(128, 128)
    ga, gb = _qgather(zemb, qa, qb)
    zd = _tcD(ga, gb)
    return zd[:, 0]


# speedup vs baseline: 1.2191x; 1.2191x over previous
import jax
import jax.numpy as jnp
from jax import lax
from jax.experimental import pallas as pl
from jax.experimental.pallas import tpu as pltpu
from jax.experimental.pallas import tpu_sc as plsc

_N = 10000
_D = 128
_E = 320000
_B = 16384
_NP = 10240
_NCH = 2560
_TCH = _NCH // 32
_RPT = _NP // 16
_RB = 2560


def _mesh():
    return plsc.VectorSubcoreMesh(
        core_axis_name="c", subcore_axis_name="s", num_cores=2, num_subcores=16
    )


def _deg_body(dstc, zvec, onev, out, idx_d, zbuf, obuf, acc):
    cid = lax.axis_index("c")
    sid = lax.axis_index("s")
    wid = sid * 2 + cid
    pltpu.sync_copy(zvec, zbuf)
    pltpu.sync_copy(zbuf, acc.at[pl.ds(sid * _RPT, _RPT)])
    pltpu.sync_copy(onev, obuf)
    plsc.subcore_barrier()
    pltpu.sync_copy(dstc.at[pl.ds(wid * _TCH, _TCH)], idx_d)

    def step(c, carry):
        pltpu.sync_copy(obuf, acc.at[idx_d.at[c]], add=True)
        return carry

    lax.fori_loop(0, _TCH, step, 0)
    plsc.subcore_barrier()
    pltpu.sync_copy(acc.at[pl.ds(sid * _RPT, _RPT)],
                    out.at[cid, pl.ds(sid * _RPT, _RPT)])


def _deg(dstc, zvec, onev):
    return pl.kernel(
        _deg_body,
        out_type=jax.ShapeDtypeStruct((2, _NP), jnp.float32),
        mesh=_mesh(),
        scratch_types=[
            pltpu.VMEM((_TCH, 128), jnp.int32),
            pltpu.VMEM((_RPT,), jnp.float32),
            pltpu.VMEM((128,), jnp.float32),
            pltpu.VMEM_SHARED((_NP,), jnp.float32),
        ],
    )(dstc, zvec, onev)


_PCH = 64
_PTC = 160
_PHH = 40
_PNB = 4
_PPD = 3


def _prop_body(table, srcc, dstc, zrows, out, idx_s, idx_d, *rest):
    bufs = rest[:_PNB]
    acc = rest[_PNB]
    sems = rest[_PNB + 1:2 * _PNB + 1]
    sss = rest[2 * _PNB + 1:]
    cid = lax.axis_index("c")
    sid = lax.axis_index("s")
    wid = sid * 2 + cid
    pltpu.sync_copy(zrows.at[pl.ds(0, _PCH)], bufs[0])
    for k in range(_RPT // _PCH):
        pltpu.sync_copy(bufs[0], acc.at[pl.ds(sid * _RPT + k * _PCH, _PCH)])
    plsc.subcore_barrier()
    for h in range(_PTC // _PHH):
        base = wid * _PTC + h * _PHH
        pltpu.sync_copy(srcc.at[pl.ds(base, _PHH)], idx_s)
        pltpu.sync_copy(dstc.at[pl.ds(base, _PHH)], idx_d)
        for b in range(_PPD):
            pltpu.async_copy(table.at[idx_s.at[b]], bufs[b], sems[b])

        def step(g, carry):
            cb = _PNB * g
            for b in range(_PNB):
                c = cb + b
                pltpu.make_async_copy(
                    table.at[idx_s.at[0]], bufs[b], sems[b]).wait()
                pltpu.async_copy(bufs[b], acc.at[idx_d.at[c]], sss[b],
                                 add=True)
                b2 = (b + _PPD) % _PNB

                @pl.when(c + _PPD < _PHH)
                def _(b2=b2, c=c):
                    @pl.when(c >= _PNB - _PPD)
                    def _():
                        pltpu.make_async_copy(
                            bufs[b2], acc.at[idx_d.at[0]], sss[b2]).wait()

                    pltpu.async_copy(table.at[idx_s.at[c + _PPD]],
                                     bufs[b2], sems[b2])

            return carry

        lax.fori_loop(0, _PHH // _PNB, step, 0)
        for b in range(_PNB):
            pltpu.make_async_copy(bufs[b], acc.at[idx_d.at[0]], sss[b]).wait()
    plsc.subcore_barrier()
    pltpu.sync_copy(acc.at[pl.ds(sid * _RPT, _RPT)],
                    out.at[cid, pl.ds(sid * _RPT, _RPT)])


def _prop(table, srcc, dstc, zrows):
    return pl.kernel(
        _prop_body,
        out_type=jax.ShapeDtypeStruct((2, _NP, _D), jnp.float32),
        mesh=_mesh(),
        scratch_types=(
            [pltpu.VMEM((_PHH, _PCH), jnp.int32),
             pltpu.VMEM((_PHH, _PCH), jnp.int32)]
            + [pltpu.VMEM((_PCH, _D), jnp.float32)] * _PNB
            + [pltpu.VMEM_SHARED((_NP, _D), jnp.float32)]
            + [pltpu.SemaphoreType.DMA] * (2 * _PNB)
        ),
    )(table, srcc, dstc, zrows)


def _qg_body(zemb, qa, qb, outa, outb, idxbuf, rows, sem):
    cid = lax.axis_index("c")
    sid = lax.axis_index("s")
    wid = sid * 2 + cid
    for qref, oref in ((qa, outa), (qb, outb)):
        pltpu.sync_copy(qref.at[pl.ds(wid * 4, 4)], idxbuf)

        def step(j, carry, oref=oref):
            pltpu.async_copy(zemb.at[idxbuf.at[j]], rows, sem).wait()
            pltpu.sync_copy(rows, oref.at[pl.ds(wid * 512 + j * 128, 128)])
            return carry

        lax.fori_loop(0, 4, step, 0)


def _qgather(zemb, qa, qb):
    return pl.kernel(
        _qg_body,
        out_type=(jax.ShapeDtypeStruct((_B, _D), jnp.float32),
                  jax.ShapeDtypeStruct((_B, _D), jnp.float32)),
        mesh=_mesh(),
        scratch_types=[
            pltpu.VMEM((4, 128), jnp.int32),
            pltpu.VMEM((128, _D), jnp.float32),
            pltpu.SemaphoreType.DMA,
        ],
    )(zemb, qa, qb)


def _tcA_body(emb_ref, p0_ref, p1_ref, w1_ref, b1_ref, w2_ref, b2_ref,
              u0_ref, dinv_ref, z2_ref):
    emb = emb_ref[...]
    deg = p0_ref[...] + p1_ref[...]
    dinv = jnp.where(deg > 0, lax.rsqrt(jnp.maximum(deg, 1e-12)), 0.0)
    u0_ref[...] = emb * dinv
    dinv_ref[...] = dinv
    h = jnp.maximum(
        jnp.dot(emb, w1_ref[...], preferred_element_type=jnp.float32)
        + b1_ref[...], 0.0)
    z2_ref[...] = jnp.maximum(
        jnp.dot(h, w2_ref[...], preferred_element_type=jnp.float32)
        + b2_ref[...], 0.0)


def _tcA(emb_p, p0, p1, w1, b1, w2, b2):
    grid = _NP // _RB
    row = pl.BlockSpec((_RB, _D), lambda i: (i, 0))
    col1 = pl.BlockSpec((_RB, 1), lambda i: (i, 0))
    wsp = pl.BlockSpec((_D, _D), lambda i: (0, 0))
    bsp = pl.BlockSpec((_D,), lambda i: (0,))
    return pl.pallas_call(
        _tcA_body,
        grid=(grid,),
        in_specs=[row, col1, col1, wsp, bsp, wsp, bsp],
        out_specs=[row, col1, row],
        out_shape=[jax.ShapeDtypeStruct((_NP, _D), jnp.float32),
                   jax.ShapeDtypeStruct((_NP, 1), jnp.float32),
                   jax.ShapeDtypeStruct((_NP, _D), jnp.float32)],
    )(emb_p, p0, p1, w1, b1, w2, b2)


def _tcB_body(t1a_ref, t1b_ref, dinv_ref, u1_ref, x1_ref):
    d = dinv_ref[...]
    x1 = d * (t1a_ref[...] + t1b_ref[...])
    x1_ref[...] = x1
    u1_ref[...] = d * x1


def _tcB(t1a, t1b, dinv):
    grid = _NP // _RB
    row = pl.BlockSpec((_RB, _D), lambda i: (i, 0))
    col1 = pl.BlockSpec((_RB, 1), lambda i: (i, 0))
    return pl.pallas_call(
        _tcB_body,
        grid=(grid,),
        in_specs=[row, row, col1],
        out_specs=[row, row],
        out_shape=[jax.ShapeDtypeStruct((_NP, _D), jnp.float32),
                   jax.ShapeDtypeStruct((_NP, _D), jnp.float32)],
    )(t1a, t1b, dinv)


def _tcC_body(emb_ref, x1_ref, t2a_ref, t2b_ref, dinv_ref, z2_ref,
              wa1_ref, ba1_ref, wa2_ref, ba2_ref, zemb_ref):
    d = dinv_ref[...]
    x2 = d * (t2a_ref[...] + t2b_ref[...])
    zp = (emb_ref[...] + x1_ref[...] + x2) * (1.0 / 3.0)
    z2 = z2_ref[...]
    wa1 = wa1_ref[...]
    ba1 = ba1_ref[...]
    wa2 = wa2_ref[...]
    ba2 = ba2_ref[...]
    s1 = jnp.dot(jnp.tanh(
        jnp.dot(zp, wa1, preferred_element_type=jnp.float32) + ba1),
        wa2, preferred_element_type=jnp.float32) + ba2
    s2 = jnp.dot(jnp.tanh(
        jnp.dot(z2, wa1, preferred_element_type=jnp.float32) + ba1),
        wa2, preferred_element_type=jnp.float32) + ba2
    s10 = s1[:, 0:1]
    s11 = s1[:, 1:2]
    s20 = s2[:, 0:1]
    s21 = s2[:, 1:2]
    m = jnp.maximum(jnp.maximum(s10, s11), jnp.maximum(s20, s21))
    e10 = jnp.exp(s10 - m)
    e11 = jnp.exp(s11 - m)
    e20 = jnp.exp(s20 - m)
    e21 = jnp.exp(s21 - m)
    tot = e10 + e11 + e20 + e21
    zemb_ref[...] = (e10 * zp + e11 * z2) / tot


def _tcC(emb_p, x1, t2a, t2b, dinv, z2, wa1, ba1, wa2p, ba2p):
    grid = _NP // _RB
    row = pl.BlockSpec((_RB, _D), lambda i: (i, 0))
    col1 = pl.BlockSpec((_RB, 1), lambda i: (i, 0))
    wsp = pl.BlockSpec((_D, _D), lambda i: (0, 0))
    bsp = pl.BlockSpec((_D,), lambda i: (0,))
    return pl.pallas_call(
        _tcC_body,
        grid=(grid,),
        in_specs=[row, row, row, row, col1, row, wsp, bsp, wsp, bsp],
        out_specs=row,
        out_shape=jax.ShapeDtypeStruct((_NP, _D), jnp.float32),
    )(emb_p, x1, t2a, t2b, dinv, z2, wa1, ba1, wa2p, ba2p)


def _tcD_body(a_ref, b_ref, o_ref):
    o_ref[...] = jnp.sum(a_ref[...] * b_ref[...], axis=1, keepdims=True)


def _tcD(ga, gb):
    rb = 2048
    grid = _B // rb
    row = pl.BlockSpec((rb, _D), lambda i: (i, 0))
    col1 = pl.BlockSpec((rb, 1), lambda i: (i, 0))
    return pl.pallas_call(
        _tcD_body,
        grid=(grid,),
        in_specs=[row, row],
        out_specs=col1,
        out_shape=jax.ShapeDtypeStruct((_B, 1), jnp.float32),
    )(ga, gb)


def kernel(edge_index, edge_label_index, emb, W1, b1, W2, b2, Wa1, ba1, Wa2, ba2):
    src = edge_index[0].astype(jnp.int32)
    dst = edge_index[1].astype(jnp.int32)
    pad = jnp.full((_NCH * 128 - _E,), _N, jnp.int32)
    srcc = jnp.concatenate([src, pad]).reshape(_NCH, 128)
    dstc = jnp.concatenate([dst, pad]).reshape(_NCH, 128)
    srcp = srcc.reshape(_NCH * 2, _PCH)
    dstp = dstc.reshape(_NCH * 2, _PCH)
    emb_p = jnp.pad(emb, ((0, _NP - _N), (0, 0)))
    zrows = jnp.zeros((128, _D), jnp.float32)
    zvec = jnp.zeros((_RPT,), jnp.float32)
    onev = jnp.ones((128,), jnp.float32)

    degp = _deg(dstc, zvec, onev)
    p0 = degp[0][:, None]
    p1 = degp[1][:, None]
    u0, dinv, z2 = _tcA(emb_p, p0, p1, W1, b1, W2, b2)
    t1 = _prop(u0, srcp, dstp, zrows)
    u1, x1 = _tcB(t1[0], t1[1], dinv)
    t2 = _prop(u1, srcp, dstp, zrows)
    wa2p = jnp.pad(Wa2, ((0, 0), (0, 126)))
    ba2p = jnp.pad(ba2, ((0, 126),))
    zemb = _tcC(emb_p, x1, t2[0], t2[1], dinv, z2, Wa1, ba1, wa2p, ba2p)
    qa = edge_label_index[0].astype(jnp.int32).reshape(128, 128)
    qb = edge_label_index[1].astype(jnp.int32).reshape(128, 128)
    ga, gb = _qgather(zemb, qa, qb)
    zd = _tcD(ga, gb)
    return zd[:, 0]


# speedup vs baseline: 1.2221x; 1.0025x over previous
import jax
import jax.numpy as jnp
from jax import lax
from jax.experimental import pallas as pl
from jax.experimental.pallas import tpu as pltpu
from jax.experimental.pallas import tpu_sc as plsc

_N = 10000
_D = 128
_E = 320000
_B = 16384
_NP = 10240
_NCH = 2560
_TCH = _NCH // 32
_RPT = _NP // 16
_RB = 2560


def _mesh():
    return plsc.VectorSubcoreMesh(
        core_axis_name="c", subcore_axis_name="s", num_cores=2, num_subcores=16
    )


def _deg_body(dstc, zvec, onev, out, idx_d, zbuf, obuf, acc):
    cid = lax.axis_index("c")
    sid = lax.axis_index("s")
    wid = sid * 2 + cid
    pltpu.sync_copy(zvec, zbuf)
    pltpu.sync_copy(zbuf, acc.at[pl.ds(sid * _RPT, _RPT)])
    pltpu.sync_copy(onev, obuf)
    plsc.subcore_barrier()
    pltpu.sync_copy(dstc.at[pl.ds(wid * _TCH, _TCH)], idx_d)

    def step(c, carry):
        pltpu.sync_copy(obuf, acc.at[idx_d.at[c]], add=True)
        return carry

    lax.fori_loop(0, _TCH, step, 0)
    plsc.subcore_barrier()
    pltpu.sync_copy(acc.at[pl.ds(sid * _RPT, _RPT)],
                    out.at[cid, pl.ds(sid * _RPT, _RPT)])


def _deg(dstc, zvec, onev):
    return pl.kernel(
        _deg_body,
        out_type=jax.ShapeDtypeStruct((2, _NP), jnp.float32),
        mesh=_mesh(),
        scratch_types=[
            pltpu.VMEM((_TCH, 128), jnp.int32),
            pltpu.VMEM((_RPT,), jnp.float32),
            pltpu.VMEM((128,), jnp.float32),
            pltpu.VMEM_SHARED((_NP,), jnp.float32),
        ],
    )(dstc, zvec, onev)


_PCH = 64
_PTC = 160
_PHH = 40
_PNB = 4
_PPD = 3


def _prop_body(table, srcc, dstc, zrows, out, idx_s, idx_d, *rest):
    bufs = rest[:_PNB]
    acc = rest[_PNB]
    sems = rest[_PNB + 1:2 * _PNB + 1]
    sss = rest[2 * _PNB + 1:]
    cid = lax.axis_index("c")
    sid = lax.axis_index("s")
    wid = sid * 2 + cid
    pltpu.sync_copy(zrows.at[pl.ds(0, _PCH)], bufs[0])
    for k in range(_RPT // _PCH):
        pltpu.sync_copy(bufs[0], acc.at[pl.ds(sid * _RPT + k * _PCH, _PCH)])
    plsc.subcore_barrier()
    for h in range(_PTC // _PHH):
        base = wid * _PTC + h * _PHH
        pltpu.sync_copy(srcc.at[pl.ds(base, _PHH)], idx_s)
        pltpu.sync_copy(dstc.at[pl.ds(base, _PHH)], idx_d)
        for b in range(_PPD):
            pltpu.async_copy(table.at[idx_s.at[b]], bufs[b], sems[b])

        def step(g, carry):
            cb = _PNB * g
            for b in range(_PNB):
                c = cb + b
                pltpu.make_async_copy(
                    table.at[idx_s.at[0]], bufs[b], sems[b]).wait()
                pltpu.async_copy(bufs[b], acc.at[idx_d.at[c]], sss[b],
                                 add=True)
                b2 = (b + _PPD) % _PNB

                @pl.when(c + _PPD < _PHH)
                def _(b2=b2, c=c):
                    @pl.when(c >= _PNB - _PPD)
                    def _():
                        pltpu.make_async_copy(
                            bufs[b2], acc.at[idx_d.at[0]], sss[b2]).wait()

                    pltpu.async_copy(table.at[idx_s.at[c + _PPD]],
                                     bufs[b2], sems[b2])

            return carry

        lax.fori_loop(0, _PHH // _PNB, step, 0)
        for b in range(_PNB):
            pltpu.make_async_copy(bufs[b], acc.at[idx_d.at[0]], sss[b]).wait()
    plsc.subcore_barrier()
    pltpu.sync_copy(acc.at[pl.ds(sid * _RPT, _RPT)],
                    out.at[cid, pl.ds(sid * _RPT, _RPT)])


def _prop(table, srcc, dstc, zrows):
    return pl.kernel(
        _prop_body,
        out_type=jax.ShapeDtypeStruct((2, _NP, _D), jnp.float32),
        mesh=_mesh(),
        scratch_types=(
            [pltpu.VMEM((_PHH, _PCH), jnp.int32),
             pltpu.VMEM((_PHH, _PCH), jnp.int32)]
            + [pltpu.VMEM((_PCH, _D), jnp.float32)] * _PNB
            + [pltpu.VMEM_SHARED((_NP, _D), jnp.float32)]
            + [pltpu.SemaphoreType.DMA] * (2 * _PNB)
        ),
    )(table, srcc, dstc, zrows)


def _qg_body(zemb, qa, qb, outa, outb, idxbuf, rows, sem):
    cid = lax.axis_index("c")
    sid = lax.axis_index("s")
    wid = sid * 2 + cid
    for qref, oref in ((qa, outa), (qb, outb)):
        pltpu.sync_copy(qref.at[pl.ds(wid * 4, 4)], idxbuf)

        def step(j, carry, oref=oref):
            pltpu.async_copy(zemb.at[idxbuf.at[j]], rows, sem).wait()
            pltpu.sync_copy(rows, oref.at[pl.ds(wid * 512 + j * 128, 128)])
            return carry

        lax.fori_loop(0, 4, step, 0)


def _qgather(zemb, qa, qb):
    return pl.kernel(
        _qg_body,
        out_type=(jax.ShapeDtypeStruct((_B, _D), jnp.float32),
                  jax.ShapeDtypeStruct((_B, _D), jnp.float32)),
        mesh=_mesh(),
        scratch_types=[
            pltpu.VMEM((4, 128), jnp.int32),
            pltpu.VMEM((128, _D), jnp.float32),
            pltpu.SemaphoreType.DMA,
        ],
    )(zemb, qa, qb)


def _tcA_body(emb_ref, p0_ref, p1_ref, u0_ref, dinv_ref):
    emb = emb_ref[...]
    deg = p0_ref[...] + p1_ref[...]
    dinv = jnp.where(deg > 0, lax.rsqrt(jnp.maximum(deg, 1e-12)), 0.0)
    u0_ref[...] = emb * dinv
    dinv_ref[...] = dinv


def _tcA(emb_p, p0, p1):
    grid = _NP // _RB
    row = pl.BlockSpec((_RB, _D), lambda i: (i, 0))
    col1 = pl.BlockSpec((_RB, 1), lambda i: (i, 0))
    return pl.pallas_call(
        _tcA_body,
        grid=(grid,),
        in_specs=[row, col1, col1],
        out_specs=[row, col1],
        out_shape=[jax.ShapeDtypeStruct((_NP, _D), jnp.float32),
                   jax.ShapeDtypeStruct((_NP, 1), jnp.float32)],
    )(emb_p, p0, p1)


def _tcA2_body(emb_ref, w1_ref, b1_ref, w2_ref, b2_ref, z2_ref):
    emb = emb_ref[...]
    h = jnp.maximum(
        jnp.dot(emb, w1_ref[...], preferred_element_type=jnp.float32)
        + b1_ref[...], 0.0)
    z2_ref[...] = jnp.maximum(
        jnp.dot(h, w2_ref[...], preferred_element_type=jnp.float32)
        + b2_ref[...], 0.0)


def _tcA2(emb_p, w1, b1, w2, b2):
    grid = _NP // _RB
    row = pl.BlockSpec((_RB, _D), lambda i: (i, 0))
    wsp = pl.BlockSpec((_D, _D), lambda i: (0, 0))
    bsp = pl.BlockSpec((_D,), lambda i: (0,))
    return pl.pallas_call(
        _tcA2_body,
        grid=(grid,),
        in_specs=[row, wsp, bsp, wsp, bsp],
        out_specs=row,
        out_shape=jax.ShapeDtypeStruct((_NP, _D), jnp.float32),
    )(emb_p, w1, b1, w2, b2)


def _tcB_body(t1a_ref, t1b_ref, dinv_ref, u1_ref, x1_ref):
    d = dinv_ref[...]
    x1 = d * (t1a_ref[...] + t1b_ref[...])
    x1_ref[...] = x1
    u1_ref[...] = d * x1


def _tcB(t1a, t1b, dinv):
    grid = _NP // _RB
    row = pl.BlockSpec((_RB, _D), lambda i: (i, 0))
    col1 = pl.BlockSpec((_RB, 1), lambda i: (i, 0))
    return pl.pallas_call(
        _tcB_body,
        grid=(grid,),
        in_specs=[row, row, col1],
        out_specs=[row, row],
        out_shape=[jax.ShapeDtypeStruct((_NP, _D), jnp.float32),
                   jax.ShapeDtypeStruct((_NP, _D), jnp.float32)],
    )(t1a, t1b, dinv)


def _tcC_body(emb_ref, x1_ref, t2a_ref, t2b_ref, dinv_ref, z2_ref,
              wa1_ref, ba1_ref, wa2_ref, ba2_ref, zemb_ref):
    d = dinv_ref[...]
    x2 = d * (t2a_ref[...] + t2b_ref[...])
    zp = (emb_ref[...] + x1_ref[...] + x2) * (1.0 / 3.0)
    z2 = z2_ref[...]
    wa1 = wa1_ref[...]
    ba1 = ba1_ref[...]
    wa2 = wa2_ref[...]
    ba2 = ba2_ref[...]
    s1 = jnp.dot(jnp.tanh(
        jnp.dot(zp, wa1, preferred_element_type=jnp.float32) + ba1),
        wa2, preferred_element_type=jnp.float32) + ba2
    s2 = jnp.dot(jnp.tanh(
        jnp.dot(z2, wa1, preferred_element_type=jnp.float32) + ba1),
        wa2, preferred_element_type=jnp.float32) + ba2
    s10 = s1[:, 0:1]
    s11 = s1[:, 1:2]
    s20 = s2[:, 0:1]
    s21 = s2[:, 1:2]
    m = jnp.maximum(jnp.maximum(s10, s11), jnp.maximum(s20, s21))
    e10 = jnp.exp(s10 - m)
    e11 = jnp.exp(s11 - m)
    e20 = jnp.exp(s20 - m)
    e21 = jnp.exp(s21 - m)
    tot = e10 + e11 + e20 + e21
    zemb_ref[...] = (e10 * zp + e11 * z2) / tot


def _tcC(emb_p, x1, t2a, t2b, dinv, z2, wa1, ba1, wa2p, ba2p):
    grid = _NP // _RB
    row = pl.BlockSpec((_RB, _D), lambda i: (i, 0))
    col1 = pl.BlockSpec((_RB, 1), lambda i: (i, 0))
    wsp = pl.BlockSpec((_D, _D), lambda i: (0, 0))
    bsp = pl.BlockSpec((_D,), lambda i: (0,))
    return pl.pallas_call(
        _tcC_body,
        grid=(grid,),
        in_specs=[row, row, row, row, col1, row, wsp, bsp, wsp, bsp],
        out_specs=row,
        out_shape=jax.ShapeDtypeStruct((_NP, _D), jnp.float32),
    )(emb_p, x1, t2a, t2b, dinv, z2, wa1, ba1, wa2p, ba2p)


def _tcD_body(a_ref, b_ref, o_ref):
    o_ref[...] = jnp.sum(a_ref[...] * b_ref[...], axis=1, keepdims=True)


def _tcD(ga, gb):
    rb = 2048
    grid = _B // rb
    row = pl.BlockSpec((rb, _D), lambda i: (i, 0))
    col1 = pl.BlockSpec((rb, 1), lambda i: (i, 0))
    return pl.pallas_call(
        _tcD_body,
        grid=(grid,),
        in_specs=[row, row],
        out_specs=col1,
        out_shape=jax.ShapeDtypeStruct((_B, 1), jnp.float32),
    )(ga, gb)


def kernel(edge_index, edge_label_index, emb, W1, b1, W2, b2, Wa1, ba1, Wa2, ba2):
    src = edge_index[0].astype(jnp.int32)
    dst = edge_index[1].astype(jnp.int32)
    pad = jnp.full((_NCH * 128 - _E,), _N, jnp.int32)
    srcc = jnp.concatenate([src, pad]).reshape(_NCH, 128)
    dstc = jnp.concatenate([dst, pad]).reshape(_NCH, 128)
    srcp = srcc.reshape(_NCH * 2, _PCH)
    dstp = dstc.reshape(_NCH * 2, _PCH)
    emb_p = jnp.pad(emb, ((0, _NP - _N), (0, 0)))
    zrows = jnp.zeros((128, _D), jnp.float32)
    zvec = jnp.zeros((_RPT,), jnp.float32)
    onev = jnp.ones((128,), jnp.float32)

    degp = _deg(dstc, zvec, onev)
    p0 = degp[0][:, None]
    p1 = degp[1][:, None]
    z2 = _tcA2(emb_p, W1, b1, W2, b2)
    u0, dinv = _tcA(emb_p, p0, p1)
    t1 = _prop(u0, srcp, dstp, zrows)
    u1, x1 = _tcB(t1[0], t1[1], dinv)
    t2 = _prop(u1, srcp, dstp, zrows)
    wa2p = jnp.pad(Wa2, ((0, 0), (0, 126)))
    ba2p = jnp.pad(ba2, ((0, 126),))
    zemb = _tcC(emb_p, x1, t2[0], t2[1], dinv, z2, Wa1, ba1, wa2p, ba2p)
    qa = edge_label_index[0].astype(jnp.int32).reshape(128, 128)
    qb = edge_label_index[1].astype(jnp.int32).reshape(128, 128)
    ga, gb = _qgather(zemb, qa, qb)
    zd = _tcD(ga, gb)
    return zd[:, 0]
